# baseline (speedup 1.0000x reference)
import functools

import jax
import jax.numpy as jnp
from jax import lax
from jax.experimental import pallas as pl
from jax.experimental.pallas import tpu as pltpu
from jax.experimental.pallas import tpu_sc as plsc

NC = 2
NS = 16
LN = 16
CHUNK_ROWS = 512


def _rsqrt_newton(d):
    i = lax.bitcast_convert_type(d, jnp.int32)
    i = jnp.int32(0x5F3759DF) - lax.shift_right_logical(i, 1)
    y = lax.bitcast_convert_type(i, jnp.float32)
    for _ in range(3):
        y = y * (jnp.float32(1.5) - jnp.float32(0.5) * d * y * y)
    return y


def _sc_build_adj(edge_index, edge_weight, n):
    e = edge_weight.shape[0]
    epw = e // NS
    ndma = epw // 128
    nloc = n // NS
    rows_per_tile = CHUNK_ROWS // NS
    words_per_tile = rows_per_tile * n
    chunks_per_core = n // CHUNK_ROWS // NC

    mesh = plsc.VectorSubcoreMesh(
        core_axis_name="c", subcore_axis_name="s",
        num_cores=NC, num_subcores=NS)

    def body(ei_hbm, ew_hbm, a_hbm,
             row_v, col_v, ew_v, val_v, sval1,
             col2, idx2,
             deg_loc, dis_loc, zbuf, obuf,
             deg_sh, a_sh, sem, sem_z):
        c = lax.axis_index("c")
        s = lax.axis_index("s")
        base = s * epw

        loads = [
            pltpu.async_copy(ei_hbm.at[0, pl.ds(base, epw)], row_v, sem),
            pltpu.async_copy(ei_hbm.at[1, pl.ds(base, epw)], col_v, sem),
            pltpu.async_copy(ew_hbm.at[pl.ds(base, epw)], ew_v, sem),
        ]
        for j in range(ndma):
            loads.append(pltpu.async_copy(
                ei_hbm.at[1, pl.ds(base + j * 128, 128)], col2.at[j], sem))

        def fill_z(i, _):
            zbuf[pl.ds(i * LN, LN)] = jnp.zeros((LN,), jnp.float32)
            return 0
        lax.fori_loop(0, zbuf.shape[0] // LN, fill_z, 0)

        def fill_o(i, _):
            obuf[pl.ds(i * LN, LN)] = jnp.ones((LN,), jnp.float32)
            return 0
        lax.fori_loop(0, nloc // LN, fill_o, 0)

        zlen = zbuf.shape[0]

        def fire_zero():
            return [pltpu.async_copy(
                zbuf, a_sh.at[pl.ds(s * words_per_tile + m * zlen, zlen)],
                sem_z) for m in range(words_per_tile // zlen)]

        zd = fire_zero()

        pltpu.sync_copy(obuf, deg_sh.at[pl.ds(s * nloc, nloc)])
        for dd in loads:
            dd.wait()
        plsc.subcore_barrier()
        degd = []
        for j in range(ndma):
            degd.append(pltpu.async_copy(
                ew_v.at[pl.ds(j * 128, 128)],
                deg_sh.at[col2.at[j]], sem, add=True))
        for dd in degd:
            dd.wait()
        plsc.subcore_barrier()

        pltpu.sync_copy(deg_sh, deg_loc)

        def dis_step(i, _):
            d = deg_loc[pl.ds(i * LN, LN)]
            dis_loc[pl.ds(i * LN, LN)] = _rsqrt_newton(d)
            return 0
        lax.fori_loop(0, n // LN, dis_step, 0)

        def val_step(i, _):
            r = row_v[pl.ds(i * LN, LN)]
            cc = col_v[pl.ds(i * LN, LN)]
            w = ew_v[pl.ds(i * LN, LN)]
            dr = plsc.load_gather(dis_loc, [r])
            dc = plsc.load_gather(dis_loc, [cc])
            val_v[pl.ds(i * LN, LN)] = dr * w * dc
            return 0
        lax.fori_loop(0, epw // LN, val_step, 0)

        iota = lax.iota(jnp.int32, LN)
        for p in range(chunks_per_core):
            lo = (c * chunks_per_core + p) * CHUNK_ROWS

            def idx_step(i, _):
                r = row_v[pl.ds(i * LN, LN)]
                cc = col_v[pl.ds(i * LN, LN)]
                v = val_v[pl.ds(i * LN, LN)]
                inr = (cc >= lo) & (cc < lo + CHUNK_ROWS)
                fi = jnp.where(inr, (cc - lo) * n + r, 0)
                j = i // (128 // LN)
                k = i % (128 // LN)
                idx2[j, pl.ds(k * LN, LN)] = fi
                sval1[pl.ds(i * LN, LN)] = jnp.where(inr, v, jnp.float32(0.0))
                return 0
            lax.fori_loop(0, epw // LN, idx_step, 0)

            for k in range(rows_per_tile // LN):
                ivec = lo + s * rows_per_tile + k * LN + iota
                dvec = plsc.load_gather(dis_loc, [ivec])
                idx2[ndma, pl.ds(k * LN, LN)] = (ivec - lo) * n + ivec
                sval1[pl.ds(epw + k * LN, LN)] = dvec * dvec
            for k in range(rows_per_tile // LN, 128 // LN):
                idx2[ndma, pl.ds(k * LN, LN)] = jnp.zeros((LN,), jnp.int32)
                sval1[pl.ds(epw + k * LN, LN)] = jnp.zeros((LN,), jnp.float32)

            for dd in zd:
                dd.wait()
            plsc.subcore_barrier()
            sd = []
            for j in range(ndma + 1):
                sd.append(pltpu.async_copy(
                    sval1.at[pl.ds(j * 128, 128)],
                    a_sh.at[idx2.at[j]], sem, add=True))
            for dd in sd:
                dd.wait()
            plsc.subcore_barrier()

            descs = []
            for r in range(rows_per_tile):
                descs.append(pltpu.async_copy(
                    a_sh.at[pl.ds((s * rows_per_tile + r) * n, n)],
                    a_hbm.at[lo + s * rows_per_tile + r], sem))
            for dd in descs:
                dd.wait()
            if p + 1 < chunks_per_core:
                zd = fire_zero()

    kern = pl.kernel(
        body,
        out_type=jax.ShapeDtypeStruct((n, n), jnp.float32),
        mesh=mesh,
        compiler_params=pltpu.CompilerParams(needs_layout_passes=False),
        scratch_types=[
            pltpu.VMEM((epw,), jnp.int32),
            pltpu.VMEM((epw,), jnp.int32),
            pltpu.VMEM((epw,), jnp.float32),
            pltpu.VMEM((epw,), jnp.float32),
            pltpu.VMEM((epw + 128,), jnp.float32),
            pltpu.VMEM((epw // 128, 128), jnp.int32),
            pltpu.VMEM((epw // 128 + 1, 128), jnp.int32),
            pltpu.VMEM((n,), jnp.float32),
            pltpu.VMEM((n,), jnp.float32),
            pltpu.VMEM((16384,), jnp.float32),
            pltpu.VMEM((nloc,), jnp.float32),
            pltpu.VMEM_SHARED((n,), jnp.float32),
            pltpu.VMEM_SHARED((CHUNK_ROWS * n,), jnp.float32),
            pltpu.SemaphoreType.DMA,
            pltpu.SemaphoreType.DMA,
        ],
    )
    return kern(edge_index, edge_weight)


def _mm_xw(x, w):
    n = x.shape[0]
    bm = 512

    def body(x_ref, w_ref, o_ref):
        o_ref[...] = jnp.dot(x_ref[...], w_ref[...],
                             preferred_element_type=jnp.float32)

    return pl.pallas_call(
        body,
        grid=(n // bm,),
        in_specs=[
            pl.BlockSpec((bm, n), lambda i: (i, 0)),
            pl.BlockSpec((n, n), lambda i: (0, 0)),
        ],
        out_specs=pl.BlockSpec((bm, n), lambda i: (i, 0)),
        out_shape=jax.ShapeDtypeStruct((n, n), jnp.float32),
    )(x, w)


def _mm_h1(a, xw, b):
    n = a.shape[0]
    bm = 512

    def body(a_ref, xw_ref, b_ref, o_ref):
        acc = jnp.dot(a_ref[...], xw_ref[...],
                      preferred_element_type=jnp.float32)
        o_ref[...] = jnp.tanh(acc + b_ref[...]).astype(jnp.bfloat16)

    return pl.pallas_call(
        body,
        grid=(n // bm,),
        in_specs=[
            pl.BlockSpec((bm, n), lambda i: (i, 0)),
            pl.BlockSpec((n, n), lambda i: (0, 0)),
            pl.BlockSpec((1, n), lambda i: (0, 0)),
        ],
        out_specs=pl.BlockSpec((bm, n), lambda i: (i, 0)),
        out_shape=jax.ShapeDtypeStruct((n, n), jnp.bfloat16),
    )(a, xw, b.reshape(1, n))


def _mm_out(h1, wq, wqb, wf, wfb):
    n = h1.shape[0]
    e = wq.shape[0]
    be = 1024

    def body(wq_ref, wqb_ref, h1_ref, wf_ref, wfb_ref, o_ref):
        m = lax.dot_general(wq_ref[...].astype(jnp.bfloat16), h1_ref[...],
                            (((1,), (1,)), ((), ())),
                            preferred_element_type=jnp.float32)
        t = jnp.tanh(m + wqb_ref[...].reshape(be, 1))
        o_ref[...] = (jnp.dot(t, wf_ref[...].reshape(n, 1),
                              preferred_element_type=jnp.float32)
                      + wfb_ref[0, 0]).reshape(1, be)

    out = pl.pallas_call(
        body,
        grid=(e // be,),
        in_specs=[
            pl.BlockSpec((be, n), lambda i: (i, 0)),
            pl.BlockSpec((1, be), lambda i: (0, i)),
            pl.BlockSpec((n, n), lambda i: (0, 0)),
            pl.BlockSpec((1, n), lambda i: (0, 0)),
            pl.BlockSpec((1, 1), lambda i: (0, 0)),
        ],
        out_specs=pl.BlockSpec((1, be), lambda i: (0, i)),
        out_shape=jax.ShapeDtypeStruct((1, e), jnp.float32),
    )(wq, wqb.reshape(1, e), h1, wf, wfb.reshape(1, 1))
    return out.reshape(e)


def kernel(x, edge_index, edge_weight, gcn_W, gcn_b, Wq_w, Wq_b, WF_w, WF_b):
    n = x.shape[0]
    a = _sc_build_adj(edge_index, edge_weight, n)
    xw = _mm_xw(x, gcn_W)
    h1 = _mm_h1(a, xw, gcn_b)
    return _mm_out(h1, Wq_w, Wq_b, WF_w, WF_b)


# speedup vs baseline: 1.0060x; 1.0060x over previous
import functools

import jax
import jax.numpy as jnp
from jax import lax
from jax.experimental import pallas as pl
from jax.experimental.pallas import tpu as pltpu
from jax.experimental.pallas import tpu_sc as plsc

NC = 2
NS = 16
LN = 16
CHUNK_ROWS = 512


def _rsqrt_newton(d):
    i = lax.bitcast_convert_type(d, jnp.int32)
    i = jnp.int32(0x5F3759DF) - lax.shift_right_logical(i, 1)
    y = lax.bitcast_convert_type(i, jnp.float32)
    for _ in range(3):
        y = y * (jnp.float32(1.5) - jnp.float32(0.5) * d * y * y)
    return y


def _sc_build_adj(edge_index, edge_weight, n):
    e = edge_weight.shape[0]
    epw = e // NS
    ndma = epw // 128
    nloc = n // NS
    rows_per_tile = CHUNK_ROWS // NS
    words_per_tile = rows_per_tile * n
    chunks_per_core = n // CHUNK_ROWS // NC

    mesh = plsc.VectorSubcoreMesh(
        core_axis_name="c", subcore_axis_name="s",
        num_cores=NC, num_subcores=NS)

    def body(ei_hbm, ew_hbm, a_hbm,
             row_v, col_v, ew_v, val_v, sval1,
             col2, idx2,
             deg_loc, dis_loc, zbuf, obuf,
             deg_sh, a_sh, sem, sem_z):
        c = lax.axis_index("c")
        s = lax.axis_index("s")
        base = s * epw

        loads = [
            pltpu.async_copy(ei_hbm.at[0, pl.ds(base, epw)], row_v, sem),
            pltpu.async_copy(ei_hbm.at[1, pl.ds(base, epw)], col_v, sem),
            pltpu.async_copy(ew_hbm.at[pl.ds(base, epw)], ew_v, sem),
        ]
        for j in range(ndma):
            loads.append(pltpu.async_copy(
                ei_hbm.at[1, pl.ds(base + j * 128, 128)], col2.at[j], sem))

        def fill_z(i, _):
            zbuf[pl.ds(i * LN, LN)] = jnp.zeros((LN,), jnp.float32)
            return 0
        lax.fori_loop(0, zbuf.shape[0] // LN, fill_z, 0)

        def fill_o(i, _):
            obuf[pl.ds(i * LN, LN)] = jnp.ones((LN,), jnp.float32)
            return 0
        lax.fori_loop(0, nloc // LN, fill_o, 0)

        zlen = zbuf.shape[0]

        def fire_zero():
            return [pltpu.async_copy(
                zbuf, a_sh.at[pl.ds(s * words_per_tile + m * zlen, zlen)],
                sem_z) for m in range(words_per_tile // zlen)]

        zd = fire_zero()

        pltpu.sync_copy(obuf, deg_sh.at[pl.ds(s * nloc, nloc)])
        for dd in loads:
            dd.wait()
        plsc.subcore_barrier()
        degd = []
        for j in range(ndma):
            degd.append(pltpu.async_copy(
                ew_v.at[pl.ds(j * 128, 128)],
                deg_sh.at[col2.at[j]], sem, add=True))
        for dd in degd:
            dd.wait()
        plsc.subcore_barrier()

        pltpu.sync_copy(deg_sh, deg_loc)

        def dis_step(i, _):
            d = deg_loc[pl.ds(i * LN, LN)]
            dis_loc[pl.ds(i * LN, LN)] = _rsqrt_newton(d)
            return 0
        lax.fori_loop(0, n // LN, dis_step, 0)

        def val_step(i, _):
            r = row_v[pl.ds(i * LN, LN)]
            cc = col_v[pl.ds(i * LN, LN)]
            w = ew_v[pl.ds(i * LN, LN)]
            dr = plsc.load_gather(dis_loc, [r])
            dc = plsc.load_gather(dis_loc, [cc])
            val_v[pl.ds(i * LN, LN)] = dr * w * dc
            return 0
        lax.fori_loop(0, epw // LN, val_step, 0)

        iota = lax.iota(jnp.int32, LN)
        for p in range(chunks_per_core):
            lo = (c * chunks_per_core + p) * CHUNK_ROWS

            def idx_step(i, _):
                r = row_v[pl.ds(i * LN, LN)]
                cc = col_v[pl.ds(i * LN, LN)]
                v = val_v[pl.ds(i * LN, LN)]
                inr = (cc >= lo) & (cc < lo + CHUNK_ROWS)
                fi = jnp.where(inr, (cc - lo) * n + r, 0)
                j = i // (128 // LN)
                k = i % (128 // LN)
                idx2[j, pl.ds(k * LN, LN)] = fi
                sval1[pl.ds(i * LN, LN)] = jnp.where(inr, v, jnp.float32(0.0))
                return 0
            lax.fori_loop(0, epw // LN, idx_step, 0)

            for k in range(rows_per_tile // LN):
                ivec = lo + s * rows_per_tile + k * LN + iota
                dvec = plsc.load_gather(dis_loc, [ivec])
                idx2[ndma, pl.ds(k * LN, LN)] = (ivec - lo) * n + ivec
                sval1[pl.ds(epw + k * LN, LN)] = dvec * dvec
            for k in range(rows_per_tile // LN, 128 // LN):
                idx2[ndma, pl.ds(k * LN, LN)] = jnp.zeros((LN,), jnp.int32)
                sval1[pl.ds(epw + k * LN, LN)] = jnp.zeros((LN,), jnp.float32)

            for dd in zd:
                dd.wait()
            plsc.subcore_barrier()
            sd = []
            for j in range(ndma + 1):
                sd.append(pltpu.async_copy(
                    sval1.at[pl.ds(j * 128, 128)],
                    a_sh.at[idx2.at[j]], sem, add=True))
            for dd in sd:
                dd.wait()
            plsc.subcore_barrier()

            descs = []
            for r in range(rows_per_tile):
                descs.append(pltpu.async_copy(
                    a_sh.at[pl.ds((s * rows_per_tile + r) * n, n)],
                    a_hbm.at[lo + s * rows_per_tile + r], sem))
            for dd in descs:
                dd.wait()
            if p + 1 < chunks_per_core:
                zd = fire_zero()

    kern = pl.kernel(
        body,
        out_type=jax.ShapeDtypeStruct((n, n), jnp.float32),
        mesh=mesh,
        compiler_params=pltpu.CompilerParams(needs_layout_passes=False),
        scratch_types=[
            pltpu.VMEM((epw,), jnp.int32),
            pltpu.VMEM((epw,), jnp.int32),
            pltpu.VMEM((epw,), jnp.float32),
            pltpu.VMEM((epw,), jnp.float32),
            pltpu.VMEM((epw + 128,), jnp.float32),
            pltpu.VMEM((epw // 128, 128), jnp.int32),
            pltpu.VMEM((epw // 128 + 1, 128), jnp.int32),
            pltpu.VMEM((n,), jnp.float32),
            pltpu.VMEM((n,), jnp.float32),
            pltpu.VMEM((16384,), jnp.float32),
            pltpu.VMEM((nloc,), jnp.float32),
            pltpu.VMEM_SHARED((n,), jnp.float32),
            pltpu.VMEM_SHARED((CHUNK_ROWS * n,), jnp.float32),
            pltpu.SemaphoreType.DMA,
            pltpu.SemaphoreType.DMA,
        ],
    )
    return kern(edge_index, edge_weight)


def _mm_xw(x, w):
    n = x.shape[0]
    bm = 512

    def body(x_ref, w_ref, o_ref):
        o_ref[...] = jnp.dot(x_ref[...], w_ref[...],
                             preferred_element_type=jnp.float32
                             ).astype(jnp.bfloat16)

    return pl.pallas_call(
        body,
        grid=(n // bm,),
        in_specs=[
            pl.BlockSpec((bm, n), lambda i: (i, 0)),
            pl.BlockSpec((n, n), lambda i: (0, 0)),
        ],
        out_specs=pl.BlockSpec((bm, n), lambda i: (i, 0)),
        out_shape=jax.ShapeDtypeStruct((n, n), jnp.bfloat16),
    )(x, w)


def _mm_h1(a, xw, b):
    n = a.shape[0]
    bm = 512

    def body(a_ref, xw_ref, b_ref, o_ref):
        acc = jnp.dot(a_ref[...], xw_ref[...],
                      preferred_element_type=jnp.float32)
        o_ref[...] = jnp.tanh(acc + b_ref[...]).astype(jnp.bfloat16)

    return pl.pallas_call(
        body,
        grid=(n // bm,),
        in_specs=[
            pl.BlockSpec((bm, n), lambda i: (i, 0)),
            pl.BlockSpec((n, n), lambda i: (0, 0)),
            pl.BlockSpec((1, n), lambda i: (0, 0)),
        ],
        out_specs=pl.BlockSpec((bm, n), lambda i: (i, 0)),
        out_shape=jax.ShapeDtypeStruct((n, n), jnp.bfloat16),
    )(a, xw, b.reshape(1, n))


def _mm_out(h1, wq, wqb, wf, wfb):
    n = h1.shape[0]
    e = wq.shape[0]
    be = 1024

    def body(wq_ref, wqb_ref, h1_ref, wf_ref, wfb_ref, o_ref):
        m = lax.dot_general(wq_ref[...].astype(jnp.bfloat16), h1_ref[...],
                            (((1,), (1,)), ((), ())),
                            preferred_element_type=jnp.float32)
        t = jnp.tanh(m + wqb_ref[...].reshape(be, 1))
        o_ref[...] = (jnp.dot(t, wf_ref[...].reshape(n, 1),
                              preferred_element_type=jnp.float32)
                      + wfb_ref[0, 0]).reshape(1, be)

    out = pl.pallas_call(
        body,
        grid=(e // be,),
        in_specs=[
            pl.BlockSpec((be, n), lambda i: (i, 0)),
            pl.BlockSpec((1, be), lambda i: (0, i)),
            pl.BlockSpec((n, n), lambda i: (0, 0)),
            pl.BlockSpec((1, n), lambda i: (0, 0)),
            pl.BlockSpec((1, 1), lambda i: (0, 0)),
        ],
        out_specs=pl.BlockSpec((1, be), lambda i: (0, i)),
        out_shape=jax.ShapeDtypeStruct((1, e), jnp.float32),
    )(wq, wqb.reshape(1, e), h1, wf, wfb.reshape(1, 1))
    return out.reshape(e)


def kernel(x, edge_index, edge_weight, gcn_W, gcn_b, Wq_w, Wq_b, WF_w, WF_b):
    n = x.shape[0]
    a = _sc_build_adj(edge_index, edge_weight, n)
    xw = _mm_xw(x, gcn_W)
    h1 = _mm_h1(a, xw, gcn_b)
    return _mm_out(h1, Wq_w, Wq_b, WF_w, WF_b)


# speedup vs baseline: 1.0153x; 1.0093x over previous
import functools

import jax
import jax.numpy as jnp
from jax import lax
from jax.experimental import pallas as pl
from jax.experimental.pallas import tpu as pltpu
from jax.experimental.pallas import tpu_sc as plsc

NC = 2
NS = 16
LN = 16
CHUNK_ROWS = 512


def _rsqrt_newton(d):
    i = lax.bitcast_convert_type(d, jnp.int32)
    i = jnp.int32(0x5F3759DF) - lax.shift_right_logical(i, 1)
    y = lax.bitcast_convert_type(i, jnp.float32)
    for _ in range(3):
        y = y * (jnp.float32(1.5) - jnp.float32(0.5) * d * y * y)
    return y


def _sc_build_adj(edge_index, edge_weight, n):
    e = edge_weight.shape[0]
    epw = e // NS
    ndma = epw // 128
    nloc = n // NS
    rows_per_tile = CHUNK_ROWS // NS
    words_per_tile = rows_per_tile * n
    chunks_per_core = n // CHUNK_ROWS // NC

    mesh = plsc.VectorSubcoreMesh(
        core_axis_name="c", subcore_axis_name="s",
        num_cores=NC, num_subcores=NS)

    def body(ei_hbm, ew_hbm, a_hbm,
             row_v, col_v, ew_v, val_v, sval1,
             col2, idx2,
             deg_loc, dis_loc, zbuf, obuf,
             deg_sh, a_sh, sem, sem_z):
        c = lax.axis_index("c")
        s = lax.axis_index("s")
        base = s * epw

        loads = [
            pltpu.async_copy(ei_hbm.at[0, pl.ds(base, epw)], row_v, sem),
            pltpu.async_copy(ei_hbm.at[1, pl.ds(base, epw)], col_v, sem),
            pltpu.async_copy(ew_hbm.at[pl.ds(base, epw)], ew_v, sem),
        ]
        for j in range(ndma):
            loads.append(pltpu.async_copy(
                ei_hbm.at[1, pl.ds(base + j * 128, 128)], col2.at[j], sem))

        @plsc.parallel_loop(0, zbuf.shape[0], step=LN, unroll=8)
        def fill_z(i):
            zbuf[pl.ds(i, LN)] = jnp.zeros((LN,), jnp.float32)

        @plsc.parallel_loop(0, nloc, step=LN, unroll=4)
        def fill_o(i):
            obuf[pl.ds(i, LN)] = jnp.ones((LN,), jnp.float32)

        zlen = zbuf.shape[0]

        def fire_zero():
            return [pltpu.async_copy(
                zbuf, a_sh.at[pl.ds(s * words_per_tile + m * zlen, zlen)],
                sem_z) for m in range(words_per_tile // zlen)]

        zd = fire_zero()

        pltpu.sync_copy(obuf, deg_sh.at[pl.ds(s * nloc, nloc)])
        for dd in loads:
            dd.wait()
        plsc.subcore_barrier()
        degd = []
        for j in range(ndma):
            degd.append(pltpu.async_copy(
                ew_v.at[pl.ds(j * 128, 128)],
                deg_sh.at[col2.at[j]], sem, add=True))
        for dd in degd:
            dd.wait()
        plsc.subcore_barrier()

        pltpu.sync_copy(deg_sh, deg_loc)

        @plsc.parallel_loop(0, n, step=LN, unroll=4)
        def dis_step(i):
            d = deg_loc[pl.ds(i, LN)]
            dis_loc[pl.ds(i, LN)] = _rsqrt_newton(d)

        @plsc.parallel_loop(0, epw, step=LN, unroll=4)
        def val_step(i):
            r = row_v[pl.ds(i, LN)]
            cc = col_v[pl.ds(i, LN)]
            w = ew_v[pl.ds(i, LN)]
            dr = plsc.load_gather(dis_loc, [r])
            dc = plsc.load_gather(dis_loc, [cc])
            val_v[pl.ds(i, LN)] = dr * w * dc

        iota = lax.iota(jnp.int32, LN)
        for p in range(chunks_per_core):
            lo = (c * chunks_per_core + p) * CHUNK_ROWS

            @plsc.parallel_loop(0, epw, step=LN, unroll=4)
            def idx_step(i):
                r = row_v[pl.ds(i, LN)]
                cc = col_v[pl.ds(i, LN)]
                v = val_v[pl.ds(i, LN)]
                inr = (cc >= lo) & (cc < lo + CHUNK_ROWS)
                fi = jnp.where(inr, (cc - lo) * n + r, 0)
                idx2[i // 128, pl.ds(i % 128, LN)] = fi
                sval1[pl.ds(i, LN)] = jnp.where(inr, v, jnp.float32(0.0))

            for k in range(rows_per_tile // LN):
                ivec = lo + s * rows_per_tile + k * LN + iota
                dvec = plsc.load_gather(dis_loc, [ivec])
                idx2[ndma, pl.ds(k * LN, LN)] = (ivec - lo) * n + ivec
                sval1[pl.ds(epw + k * LN, LN)] = dvec * dvec
            for k in range(rows_per_tile // LN, 128 // LN):
                idx2[ndma, pl.ds(k * LN, LN)] = jnp.zeros((LN,), jnp.int32)
                sval1[pl.ds(epw + k * LN, LN)] = jnp.zeros((LN,), jnp.float32)

            for dd in zd:
                dd.wait()
            plsc.subcore_barrier()
            sd = []
            for j in range(ndma + 1):
                sd.append(pltpu.async_copy(
                    sval1.at[pl.ds(j * 128, 128)],
                    a_sh.at[idx2.at[j]], sem, add=True))
            for dd in sd:
                dd.wait()
            plsc.subcore_barrier()

            descs = []
            for r in range(rows_per_tile):
                descs.append(pltpu.async_copy(
                    a_sh.at[pl.ds((s * rows_per_tile + r) * n, n)],
                    a_hbm.at[lo + s * rows_per_tile + r], sem))
            for dd in descs:
                dd.wait()
            if p + 1 < chunks_per_core:
                zd = fire_zero()

    kern = pl.kernel(
        body,
        out_type=jax.ShapeDtypeStruct((n, n), jnp.float32),
        mesh=mesh,
        compiler_params=pltpu.CompilerParams(needs_layout_passes=False),
        scratch_types=[
            pltpu.VMEM((epw,), jnp.int32),
            pltpu.VMEM((epw,), jnp.int32),
            pltpu.VMEM((epw,), jnp.float32),
            pltpu.VMEM((epw,), jnp.float32),
            pltpu.VMEM((epw + 128,), jnp.float32),
            pltpu.VMEM((epw // 128, 128), jnp.int32),
            pltpu.VMEM((epw // 128 + 1, 128), jnp.int32),
            pltpu.VMEM((n,), jnp.float32),
            pltpu.VMEM((n,), jnp.float32),
            pltpu.VMEM((2048,), jnp.float32),
            pltpu.VMEM((nloc,), jnp.float32),
            pltpu.VMEM_SHARED((n,), jnp.float32),
            pltpu.VMEM_SHARED((CHUNK_ROWS * n,), jnp.float32),
            pltpu.SemaphoreType.DMA,
            pltpu.SemaphoreType.DMA,
        ],
    )
    return kern(edge_index, edge_weight)


def _mm_xw(x, w):
    n = x.shape[0]
    bm = 512

    def body(x_ref, w_ref, o_ref):
        o_ref[...] = jnp.dot(x_ref[...], w_ref[...],
                             preferred_element_type=jnp.float32
                             ).astype(jnp.bfloat16)

    return pl.pallas_call(
        body,
        grid=(n // bm,),
        in_specs=[
            pl.BlockSpec((bm, n), lambda i: (i, 0)),
            pl.BlockSpec((n, n), lambda i: (0, 0)),
        ],
        out_specs=pl.BlockSpec((bm, n), lambda i: (i, 0)),
        out_shape=jax.ShapeDtypeStruct((n, n), jnp.bfloat16),
    )(x, w)


def _mm_h1(a, xw, b):
    n = a.shape[0]
    bm = 512

    def body(a_ref, xw_ref, b_ref, o_ref):
        acc = jnp.dot(a_ref[...], xw_ref[...],
                      preferred_element_type=jnp.float32)
        o_ref[...] = jnp.tanh(acc + b_ref[...]).astype(jnp.bfloat16)

    return pl.pallas_call(
        body,
        grid=(n // bm,),
        in_specs=[
            pl.BlockSpec((bm, n), lambda i: (i, 0)),
            pl.BlockSpec((n, n), lambda i: (0, 0)),
            pl.BlockSpec((1, n), lambda i: (0, 0)),
        ],
        out_specs=pl.BlockSpec((bm, n), lambda i: (i, 0)),
        out_shape=jax.ShapeDtypeStruct((n, n), jnp.bfloat16),
    )(a, xw, b.reshape(1, n))


def _mm_out(h1, wq, wqb, wf, wfb):
    n = h1.shape[0]
    e = wq.shape[0]
    be = 1024

    def body(wq_ref, wqb_ref, h1_ref, wf_ref, wfb_ref, o_ref):
        m = lax.dot_general(wq_ref[...].astype(jnp.bfloat16), h1_ref[...],
                            (((1,), (1,)), ((), ())),
                            preferred_element_type=jnp.float32)
        t = jnp.tanh(m + wqb_ref[...].reshape(be, 1))
        o_ref[...] = (jnp.dot(t, wf_ref[...].reshape(n, 1),
                              preferred_element_type=jnp.float32)
                      + wfb_ref[0, 0]).reshape(1, be)

    out = pl.pallas_call(
        body,
        grid=(e // be,),
        in_specs=[
            pl.BlockSpec((be, n), lambda i: (i, 0)),
            pl.BlockSpec((1, be), lambda i: (0, i)),
            pl.BlockSpec((n, n), lambda i: (0, 0)),
            pl.BlockSpec((1, n), lambda i: (0, 0)),
            pl.BlockSpec((1, 1), lambda i: (0, 0)),
        ],
        out_specs=pl.BlockSpec((1, be), lambda i: (0, i)),
        out_shape=jax.ShapeDtypeStruct((1, e), jnp.float32),
    )(wq, wqb.reshape(1, e), h1, wf, wfb.reshape(1, 1))
    return out.reshape(e)


def kernel(x, edge_index, edge_weight, gcn_W, gcn_b, Wq_w, Wq_b, WF_w, WF_b):
    n = x.shape[0]
    a = _sc_build_adj(edge_index, edge_weight, n)
    xw = _mm_xw(x, gcn_W)
    h1 = _mm_h1(a, xw, gcn_b)
    return _mm_out(h1, Wq_w, Wq_b, WF_w, WF_b)


# speedup vs baseline: 1.0166x; 1.0012x over previous
import functools

import jax
import jax.numpy as jnp
from jax import lax
from jax.experimental import pallas as pl
from jax.experimental.pallas import tpu as pltpu
from jax.experimental.pallas import tpu_sc as plsc

NC = 2
NS = 16
LN = 16
CHUNK_ROWS = 512


def _rsqrt_newton(d):
    i = lax.bitcast_convert_type(d, jnp.int32)
    i = jnp.int32(0x5F3759DF) - lax.shift_right_logical(i, 1)
    y = lax.bitcast_convert_type(i, jnp.float32)
    for _ in range(3):
        y = y * (jnp.float32(1.5) - jnp.float32(0.5) * d * y * y)
    return y


def _sc_build_adj(edge_index, edge_weight, n):
    e = edge_weight.shape[0]
    epw = e // NS
    ndma = epw // 128
    nloc = n // NS
    rows_per_tile = CHUNK_ROWS // NS
    words_per_tile = rows_per_tile * n
    chunks_per_core = n // CHUNK_ROWS // NC

    mesh = plsc.VectorSubcoreMesh(
        core_axis_name="c", subcore_axis_name="s",
        num_cores=NC, num_subcores=NS)

    def body(ei_hbm, ew_hbm, a_hbm,
             row_v, col_v, ew_v, val_v, sval1,
             col2, idx2,
             deg_loc, dis_loc, zbuf, obuf,
             deg_sh, a_sh, sem, sem_z):
        c = lax.axis_index("c")
        s = lax.axis_index("s")
        base = s * epw

        loads = [
            pltpu.async_copy(ei_hbm.at[0, pl.ds(base, epw)], row_v, sem),
            pltpu.async_copy(ei_hbm.at[1, pl.ds(base, epw)], col_v, sem),
            pltpu.async_copy(ew_hbm.at[pl.ds(base, epw)], ew_v, sem),
        ]
        for j in range(ndma):
            loads.append(pltpu.async_copy(
                ei_hbm.at[1, pl.ds(base + j * 128, 128)], col2.at[j], sem))

        @plsc.parallel_loop(0, zbuf.shape[0], step=LN, unroll=8)
        def fill_z(i):
            zbuf[pl.ds(i, LN)] = jnp.zeros((LN,), jnp.float32)

        @plsc.parallel_loop(0, nloc, step=LN, unroll=4)
        def fill_o(i):
            obuf[pl.ds(i, LN)] = jnp.ones((LN,), jnp.float32)

        zlen = zbuf.shape[0]

        def fire_zero():
            return [pltpu.async_copy(
                zbuf, a_sh.at[pl.ds(s * words_per_tile + m * zlen, zlen)],
                sem_z) for m in range(words_per_tile // zlen)]

        zd = fire_zero()

        pltpu.sync_copy(obuf, deg_sh.at[pl.ds(s * nloc, nloc)])
        for dd in loads:
            dd.wait()
        plsc.subcore_barrier()
        degd = []
        for j in range(ndma):
            degd.append(pltpu.async_copy(
                ew_v.at[pl.ds(j * 128, 128)],
                deg_sh.at[col2.at[j]], sem, add=True))
        for dd in degd:
            dd.wait()
        plsc.subcore_barrier()

        pltpu.sync_copy(deg_sh, deg_loc)

        @plsc.parallel_loop(0, n, step=LN, unroll=4)
        def dis_step(i):
            d = deg_loc[pl.ds(i, LN)]
            dis_loc[pl.ds(i, LN)] = _rsqrt_newton(d)

        @plsc.parallel_loop(0, epw, step=LN, unroll=4)
        def val_step(i):
            r = row_v[pl.ds(i, LN)]
            cc = col_v[pl.ds(i, LN)]
            w = ew_v[pl.ds(i, LN)]
            dr = plsc.load_gather(dis_loc, [r])
            dc = plsc.load_gather(dis_loc, [cc])
            val_v[pl.ds(i, LN)] = dr * w * dc

        iota = lax.iota(jnp.int32, LN)
        for p in range(chunks_per_core):
            lo = (c * chunks_per_core + p) * CHUNK_ROWS

            @plsc.parallel_loop(0, epw, step=LN, unroll=4)
            def idx_step(i):
                r = row_v[pl.ds(i, LN)]
                cc = col_v[pl.ds(i, LN)]
                v = val_v[pl.ds(i, LN)]
                inr = (cc >= lo) & (cc < lo + CHUNK_ROWS)
                fi = jnp.where(inr, (cc - lo) * n + r, 0)
                idx2[i // 128, pl.ds(i % 128, LN)] = fi
                sval1[pl.ds(i, LN)] = jnp.where(inr, v, jnp.float32(0.0))

            for k in range(rows_per_tile // LN):
                ivec = lo + s * rows_per_tile + k * LN + iota
                dvec = plsc.load_gather(dis_loc, [ivec])
                idx2[ndma, pl.ds(k * LN, LN)] = (ivec - lo) * n + ivec
                sval1[pl.ds(epw + k * LN, LN)] = dvec * dvec
            for k in range(rows_per_tile // LN, 128 // LN):
                idx2[ndma, pl.ds(k * LN, LN)] = jnp.zeros((LN,), jnp.int32)
                sval1[pl.ds(epw + k * LN, LN)] = jnp.zeros((LN,), jnp.float32)

            for dd in zd:
                dd.wait()
            plsc.subcore_barrier()
            sd = []
            for j in range(ndma + 1):
                sd.append(pltpu.async_copy(
                    sval1.at[pl.ds(j * 128, 128)],
                    a_sh.at[idx2.at[j]], sem, add=True))
            for dd in sd:
                dd.wait()
            plsc.subcore_barrier()

            descs = []
            for r in range(rows_per_tile):
                descs.append(pltpu.async_copy(
                    a_sh.at[pl.ds((s * rows_per_tile + r) * n, n)],
                    a_hbm.at[lo + s * rows_per_tile + r], sem))
            for dd in descs:
                dd.wait()
            if p + 1 < chunks_per_core:
                zd = fire_zero()

    kern = pl.kernel(
        body,
        out_type=jax.ShapeDtypeStruct((n, n), jnp.float32),
        mesh=mesh,
        compiler_params=pltpu.CompilerParams(needs_layout_passes=False),
        scratch_types=[
            pltpu.VMEM((epw,), jnp.int32),
            pltpu.VMEM((epw,), jnp.int32),
            pltpu.VMEM((epw,), jnp.float32),
            pltpu.VMEM((epw,), jnp.float32),
            pltpu.VMEM((epw + 128,), jnp.float32),
            pltpu.VMEM((epw // 128, 128), jnp.int32),
            pltpu.VMEM((epw // 128 + 1, 128), jnp.int32),
            pltpu.VMEM((n,), jnp.float32),
            pltpu.VMEM((n,), jnp.float32),
            pltpu.VMEM((2048,), jnp.float32),
            pltpu.VMEM((nloc,), jnp.float32),
            pltpu.VMEM_SHARED((n,), jnp.float32),
            pltpu.VMEM_SHARED((CHUNK_ROWS * n,), jnp.float32),
            pltpu.SemaphoreType.DMA,
            pltpu.SemaphoreType.DMA,
        ],
    )
    return kern(edge_index, edge_weight)


def _mm_xw(x, w):
    n = x.shape[0]
    bm = 512

    def body(x_ref, w_ref, o_ref):
        o_ref[...] = jnp.dot(x_ref[...], w_ref[...],
                             preferred_element_type=jnp.float32
                             ).astype(jnp.bfloat16)

    return pl.pallas_call(
        body,
        grid=(n // bm,),
        in_specs=[
            pl.BlockSpec((bm, n), lambda i: (i, 0)),
            pl.BlockSpec((n, n), lambda i: (0, 0)),
        ],
        out_specs=pl.BlockSpec((bm, n), lambda i: (i, 0)),
        out_shape=jax.ShapeDtypeStruct((n, n), jnp.bfloat16),
    )(x, w)


def _mm_h1(a, xw, b):
    n = a.shape[0]
    bm = 256

    def body(a_ref, xw_ref, b_ref, o_ref):
        acc = jnp.dot(a_ref[...], xw_ref[...],
                      preferred_element_type=jnp.float32)
        o_ref[...] = jnp.tanh(acc + b_ref[...]).astype(jnp.bfloat16)

    return pl.pallas_call(
        body,
        grid=(n // bm,),
        in_specs=[
            pl.BlockSpec((bm, n), lambda i: (i, 0)),
            pl.BlockSpec((n, n), lambda i: (0, 0)),
            pl.BlockSpec((1, n), lambda i: (0, 0)),
        ],
        out_specs=pl.BlockSpec((bm, n), lambda i: (i, 0)),
        out_shape=jax.ShapeDtypeStruct((n, n), jnp.bfloat16),
    )(a, xw, b.reshape(1, n))


def _mm_out(h1, wq, wqb, wf, wfb):
    n = h1.shape[0]
    e = wq.shape[0]
    be = 1024

    def body(wq_ref, wqb_ref, h1_ref, wf_ref, wfb_ref, o_ref):
        m = lax.dot_general(wq_ref[...].astype(jnp.bfloat16), h1_ref[...],
                            (((1,), (1,)), ((), ())),
                            preferred_element_type=jnp.float32)
        t = jnp.tanh(m + wqb_ref[...].reshape(be, 1))
        o_ref[...] = (jnp.dot(t, wf_ref[...].reshape(n, 1),
                              preferred_element_type=jnp.float32)
                      + wfb_ref[0, 0]).reshape(1, be)

    out = pl.pallas_call(
        body,
        grid=(e // be,),
        in_specs=[
            pl.BlockSpec((be, n), lambda i: (i, 0)),
            pl.BlockSpec((1, be), lambda i: (0, i)),
            pl.BlockSpec((n, n), lambda i: (0, 0)),
            pl.BlockSpec((1, n), lambda i: (0, 0)),
            pl.BlockSpec((1, 1), lambda i: (0, 0)),
        ],
        out_specs=pl.BlockSpec((1, be), lambda i: (0, i)),
        out_shape=jax.ShapeDtypeStruct((1, e), jnp.float32),
    )(wq, wqb.reshape(1, e), h1, wf, wfb.reshape(1, 1))
    return out.reshape(e)


def kernel(x, edge_index, edge_weight, gcn_W, gcn_b, Wq_w, Wq_b, WF_w, WF_b):
    n = x.shape[0]
    a = _sc_build_adj(edge_index, edge_weight, n)
    xw = _mm_xw(x, gcn_W)
    h1 = _mm_h1(a, xw, gcn_b)
    return _mm_out(h1, Wq_w, Wq_b, WF_w, WF_b)
